# baseline (speedup 1.0000x reference)
import functools
import jax
import jax.numpy as jnp
import numpy as np
from jax.experimental import pallas as pl
from jax.experimental.pallas import tpu as pltpu

N_EXP = 64
D_MODEL = 8
D_FF = 16
BLK = 2048


def _moe_block(x_ref, M2_ref, Wg_ref, Wu_ref, Wd_ref, G1_ref, G2_ref,
               S8q_ref, S16_ref, Rx_ref, Rh2_ref, Wsd2_ref, o_ref):
    f32 = jnp.float32
    dot = functools.partial(jnp.dot, preferred_element_type=f32)
    x = x_ref[...]
    t1 = dot(x, M2_ref[...])
    logits = t1[:, :N_EXP]

    m = jnp.max(logits, axis=1, keepdims=True)
    d = logits - m
    eq = (d == 0.0).astype(f32)
    e1 = jnp.exp(d)
    aux = dot(eq, G1_ref[...]) + dot(e1, G2_ref[...])
    oh = jnp.where(aux[:, :N_EXP] > 0.5, 0.0, eq)
    ohw = oh / aux[:, N_EXP:N_EXP + 1]

    ohb = oh.astype(jnp.bfloat16)
    Wtg = dot(ohb, Wg_ref[...])
    Wtu = dot(ohb, Wu_ref[...])
    Wtd = dot(ohw.astype(jnp.bfloat16), Wd_ref[...])

    xt = dot(x, Rx_ref[...])
    q = dot(Wtg * xt, S8q_ref[...]) + t1[:, N_EXP:N_EXP + 32]
    r = dot(Wtu * xt, S8q_ref[...]) + t1[:, N_EXP + 32:]
    hh = (q * jax.nn.sigmoid(q)) * r

    ht = dot(hh, Rh2_ref[...])
    routed = dot(Wtd * ht, S16_ref[...])
    o_ref[...] = routed + dot(hh, Wsd2_ref[...])


def kernel(hidden_states, gate_weight, Wg, Wu, Wd, Wsg, Wsu, Wsd):
    Bsz, S, D = hidden_states.shape
    T = Bsz * S
    x2 = hidden_states.reshape(T, D)

    Z16 = jnp.zeros((D_MODEL, D_FF), jnp.float32)
    M2 = jnp.concatenate([gate_weight.T, Z16, Wsg.T, Z16, Wsu.T], axis=1)
    G1 = np.zeros((N_EXP, N_EXP + D_MODEL), np.float32)
    G1[:, :N_EXP] = np.tril(np.ones((N_EXP, N_EXP), np.float32), k=-1).T
    G2 = np.zeros((N_EXP, N_EXP + D_MODEL), np.float32)
    G2[:, N_EXP:] = 1.0
    S8 = np.kron(np.eye(D_FF, dtype=np.float32), np.ones((D_MODEL, 1), np.float32))
    S8q = np.zeros((128, 32), np.float32)
    S8q[:, :D_FF] = S8
    S16 = np.kron(np.eye(D_MODEL, dtype=np.float32), np.ones((D_FF, 1), np.float32))
    Rx = np.kron(np.ones((1, D_FF), np.float32), np.eye(D_MODEL, dtype=np.float32))
    Rh2 = np.zeros((32, 128), np.float32)
    Rh2[:D_FF] = np.kron(np.ones((1, D_MODEL), np.float32),
                         np.eye(D_FF, dtype=np.float32))
    G1, G2, S8q, S16, Rx, Rh2 = map(jnp.asarray, (G1, G2, S8q, S16, Rx, Rh2))
    Wsd2 = jnp.concatenate([jnp.zeros((D_FF, D_MODEL), jnp.float32), Wsd.T],
                           axis=0)

    full = lambda arr: pl.BlockSpec(arr.shape, lambda i: (0,) * arr.ndim)
    bf16 = jnp.bfloat16
    args = (x2, M2, Wg.reshape(N_EXP, 128).astype(bf16),
            Wu.reshape(N_EXP, 128).astype(bf16),
            Wd.reshape(N_EXP, 128).astype(bf16), G1, G2, S8q, S16, Rx, Rh2,
            Wsd2)
    out = pl.pallas_call(
        _moe_block,
        grid=(T // BLK,),
        in_specs=[pl.BlockSpec((BLK, D_MODEL), lambda i: (i, 0))]
        + [full(a) for a in args[1:]],
        out_specs=pl.BlockSpec((BLK, D_MODEL), lambda i: (i, 0)),
        out_shape=jax.ShapeDtypeStruct((T, D_MODEL), jnp.float32),
        compiler_params=pltpu.CompilerParams(dimension_semantics=("parallel",)),
    )(*args)
    return out.reshape(Bsz, S, D)


# speedup vs baseline: 1.0005x; 1.0005x over previous
import functools
import jax
import jax.numpy as jnp
import numpy as np
from jax.experimental import pallas as pl
from jax.experimental.pallas import tpu as pltpu

N_EXP = 64
D_MODEL = 8
D_FF = 16
BLK = 2048


def _moe_block(x_ref, M2_ref, Wg_ref, Wu_ref, Wd_ref, G1_ref, G2_ref,
               S8q_ref, S16_ref, Rx_ref, Rh2_ref, Wsd2_ref, o_ref):
    f32 = jnp.float32
    dot = functools.partial(jnp.dot, preferred_element_type=f32)
    x = x_ref[...]
    t1 = dot(x, M2_ref[...])
    logits = t1[:, :N_EXP]

    m = jnp.max(logits, axis=1, keepdims=True)
    d = logits - m
    eq = (d == 0.0).astype(f32)
    e1 = jnp.exp(d)
    aux = dot(eq, G1_ref[...]) + dot(e1, G2_ref[...])
    oh = jnp.where(aux[:, :N_EXP] > 0.5, 0.0, eq)
    ohw = oh / aux[:, N_EXP:N_EXP + 1]

    Wtg = dot(oh, Wg_ref[...])
    Wtu = dot(oh, Wu_ref[...])
    Wtd = dot(ohw, Wd_ref[...])

    xt = dot(x, Rx_ref[...])
    q = dot(Wtg * xt, S8q_ref[...]) + t1[:, N_EXP:N_EXP + 32]
    r = dot(Wtu * xt, S8q_ref[...]) + t1[:, N_EXP + 32:]
    hh = (q * jax.nn.sigmoid(q)) * r

    ht = dot(hh, Rh2_ref[...])
    routed = dot(Wtd * ht, S16_ref[...])
    o_ref[...] = routed + dot(hh, Wsd2_ref[...])


def kernel(hidden_states, gate_weight, Wg, Wu, Wd, Wsg, Wsu, Wsd):
    Bsz, S, D = hidden_states.shape
    T = Bsz * S
    x2 = hidden_states.reshape(T, D)

    Z16 = jnp.zeros((D_MODEL, D_FF), jnp.float32)
    M2 = jnp.concatenate([gate_weight.T, Z16, Wsg.T, Z16, Wsu.T], axis=1)
    G1 = np.zeros((N_EXP, N_EXP + D_MODEL), np.float32)
    G1[:, :N_EXP] = np.tril(np.ones((N_EXP, N_EXP), np.float32), k=-1).T
    G2 = np.zeros((N_EXP, N_EXP + D_MODEL), np.float32)
    G2[:, N_EXP:] = 1.0
    S8 = np.kron(np.eye(D_FF, dtype=np.float32), np.ones((D_MODEL, 1), np.float32))
    S8q = np.zeros((128, 32), np.float32)
    S8q[:, :D_FF] = S8
    S16 = np.kron(np.eye(D_MODEL, dtype=np.float32), np.ones((D_FF, 1), np.float32))
    Rx = np.kron(np.ones((1, D_FF), np.float32), np.eye(D_MODEL, dtype=np.float32))
    Rh2 = np.zeros((32, 128), np.float32)
    Rh2[:D_FF] = np.kron(np.ones((1, D_MODEL), np.float32),
                         np.eye(D_FF, dtype=np.float32))
    G1, G2, S8q, S16, Rx, Rh2 = map(jnp.asarray, (G1, G2, S8q, S16, Rx, Rh2))
    Wsd2 = jnp.concatenate([jnp.zeros((D_FF, D_MODEL), jnp.float32), Wsd.T],
                           axis=0)

    full = lambda arr: pl.BlockSpec(arr.shape, lambda i: (0,) * arr.ndim)
    args = (x2, M2, Wg.reshape(N_EXP, 128), Wu.reshape(N_EXP, 128),
            Wd.reshape(N_EXP, 128), G1, G2, S8q, S16, Rx, Rh2, Wsd2)
    out = pl.pallas_call(
        _moe_block,
        grid=(T // BLK,),
        in_specs=[pl.BlockSpec((BLK, D_MODEL), lambda i: (i, 0))]
        + [full(a) for a in args[1:]],
        out_specs=pl.BlockSpec((BLK, D_MODEL), lambda i: (i, 0)),
        out_shape=jax.ShapeDtypeStruct((T, D_MODEL), jnp.float32),
        compiler_params=pltpu.CompilerParams(dimension_semantics=("parallel",)),
    )(*args)
    return out.reshape(Bsz, S, D)
